# baseline (speedup 1.0000x reference)
import jax
import jax.numpy as jnp
from jax import lax
from jax.experimental import pallas as pl
from jax.experimental.pallas import tpu as pltpu
from jax.experimental.pallas import tpu_sc as plsc

_N, _E, _D, _R, _NB, _G = 10000, 320000, 128, 8, 30, 64
_NC, _NS, _L = 2, 16, 16
_NW = _NC * _NS
_C = 64
_EP = 327680
_ECP = _EP // _C
_RPW = _ECP // _NW
_RPS = _ECP // _NS
_NACC = 10112
_ZR = _NACC // _NS
_CNTSZ = _NACC * _R
_NRS = _CNTSZ // _NS
_SS = 32
_BN = 1000

_mesh = plsc.VectorSubcoreMesh(core_axis_name="c", subcore_axis_name="s",
                               num_cores=_NC, num_subcores=_NS)


def _wcomb_body(comp_ref, basis_ref, out_ref):
    out_ref[...] = jnp.dot(comp_ref[...], basis_ref[...],
                           preferred_element_type=jnp.float32)


def _combine_weights(comp, basis):
    wfull = pl.pallas_call(
        _wcomb_body,
        out_shape=jax.ShapeDtypeStruct((_R, _D * _D), jnp.float32),
    )(comp, basis.reshape(_NB, _D * _D))
    return wfull.reshape(_R, _D, _D).transpose(1, 0, 2).reshape(_D, _R * _D)


def _mm_body(x_ref, wcat_ref, root_ref, xw_ref, xr_ref):
    xb = x_ref[...].astype(jnp.bfloat16)
    xw_ref[...] = jnp.dot(xb, wcat_ref[...].astype(jnp.bfloat16),
                          preferred_element_type=jnp.float32
                          ).astype(jnp.bfloat16)

    @pl.when(pl.program_id(1) == 0)
    def _():
        xr_ref[...] = jnp.dot(xb, root_ref[...].astype(jnp.bfloat16),
                              preferred_element_type=jnp.float32)


def _mm(x, wcat, root):
    return pl.pallas_call(
        _mm_body,
        grid=(_N // _BN, _R),
        in_specs=[pl.BlockSpec((_BN, _D), lambda i, r: (i, 0)),
                  pl.BlockSpec((_D, _D), lambda i, r: (0, r)),
                  pl.BlockSpec((_D, _D), lambda i, r: (0, 0))],
        out_specs=[pl.BlockSpec((_BN, _D),
                                lambda i, r: (r * (_N // _BN) + i, 0)),
                   pl.BlockSpec((_BN, _D), lambda i, r: (i, 0))],
        out_shape=[jax.ShapeDtypeStruct((_R * _N, _D), jnp.bfloat16),
                   jax.ShapeDtypeStruct((_N, _D), jnp.float32)],
    )(x, wcat, root)


def _comb_body(p_ref, xr_ref, b_ref, wcat_ref, root_ref,
               h_ref, xw_ref, xr2_ref, x2_ref):
    @pl.when(pl.program_id(1) == 0)
    def _():
        h = p_ref[0] + p_ref[1] + xr_ref[...] + b_ref[...]
        h_ref[...] = h
        x2 = jnp.maximum(h, 0.0).astype(jnp.bfloat16)
        x2_ref[...] = x2
        xr2_ref[...] = jnp.dot(x2, root_ref[...].astype(jnp.bfloat16),
                               preferred_element_type=jnp.float32)

    xw_ref[...] = jnp.dot(x2_ref[...], wcat_ref[...].astype(jnp.bfloat16),
                          preferred_element_type=jnp.float32
                          ).astype(jnp.bfloat16)


def _combine_relu_mm(part, xr, bias2d, wcat, root):
    return pl.pallas_call(
        _comb_body,
        grid=(_N // _BN, _R),
        in_specs=[pl.BlockSpec((_NC, _BN, _D), lambda i, r: (0, i, 0)),
                  pl.BlockSpec((_BN, _D), lambda i, r: (i, 0)),
                  pl.BlockSpec((1, _D), lambda i, r: (0, 0)),
                  pl.BlockSpec((_D, _D), lambda i, r: (0, r)),
                  pl.BlockSpec((_D, _D), lambda i, r: (0, 0))],
        out_specs=[pl.BlockSpec((_BN, _D), lambda i, r: (i, 0)),
                   pl.BlockSpec((_BN, _D),
                                lambda i, r: (r * (_N // _BN) + i, 0)),
                   pl.BlockSpec((_BN, _D), lambda i, r: (i, 0))],
        out_shape=[jax.ShapeDtypeStruct((_N, _D), jnp.float32),
                   jax.ShapeDtypeStruct((_R * _N, _D), jnp.bfloat16),
                   jax.ShapeDtypeStruct((_N, _D), jnp.float32)],
        scratch_shapes=[pltpu.VMEM((_BN, _D), jnp.bfloat16)],
    )(part, xr, bias2d, wcat, root)


def _final_body(p_ref, xr_ref, b_ref, bt_ref, h_ref, gf_ref):
    h = p_ref[0] + p_ref[1] + xr_ref[...] + b_ref[...]
    h_ref[...] = h
    gids = lax.broadcasted_iota(jnp.int32, (1, _G), 1)
    onehot = (bt_ref[...] == gids).astype(jnp.float32)
    contrib = lax.dot_general(onehot, h, (((0,), (0,)), ((), ())),
                              preferred_element_type=jnp.float32)

    @pl.when(pl.program_id(0) == 0)
    def _():
        gf_ref[...] = contrib

    @pl.when(pl.program_id(0) != 0)
    def _():
        gf_ref[...] += contrib


def _combine_pool(part, xr, bias2d, batch2d):
    return pl.pallas_call(
        _final_body,
        grid=(_N // _BN,),
        in_specs=[pl.BlockSpec((_NC, _BN, _D), lambda i: (0, i, 0)),
                  pl.BlockSpec((_BN, _D), lambda i: (i, 0)),
                  pl.BlockSpec((1, _D), lambda i: (0, 0)),
                  pl.BlockSpec((_BN, 1), lambda i: (i, 0))],
        out_specs=[pl.BlockSpec((_BN, _D), lambda i: (i, 0)),
                   pl.BlockSpec((_G, _D), lambda i: (0, 0))],
        out_shape=[jax.ShapeDtypeStruct((_N, _D), jnp.float32),
                   jax.ShapeDtypeStruct((_G, _D), jnp.float32)],
    )(part, xr, bias2d, batch2d)


def _edge_prep_body(src_hbm, dst_hbm, rt_hbm,
                    gidx_hbm, scale_hbm,
                    cnt_sh, dstA, rtA, segA, srcB, scaleB, gidxB,
                    cntrow, ones_v, zbuf, sem):
    c = lax.axis_index("c")
    s = lax.axis_index("s")
    w = s * _NC + c

    def zfill(j, carry):
        zbuf[pl.ds(j * _L, _L)] = jnp.zeros((_L,), jnp.float32)
        return carry

    lax.fori_loop(0, _NRS // _L, zfill, 0)
    pltpu.sync_copy(zbuf, cnt_sh.at[pl.ds(s * _NRS, _NRS)])
    pltpu.sync_copy(dst_hbm.at[pl.ds(s * _RPS, _RPS)], dstA)
    pltpu.sync_copy(rt_hbm.at[pl.ds(s * _RPS, _RPS)], rtA)
    for u in range(_C // _L):
        ones_v[pl.ds(u * _L, _L)] = jnp.full((_L,), 1.0, jnp.float32)
    plsc.subcore_barrier()

    def cbody(j, carry):
        for u in range(8):
            row = j * 8 + u
            for g in range(_C // _L):
                sl = (row, pl.ds(g * _L, _L))
                segA[sl] = dstA[sl] * _R + rtA[sl]
            pltpu.async_copy(ones_v, cnt_sh.at[segA.at[row]], sem, add=True)
        for u in range(8):
            pltpu.make_async_copy(ones_v, cnt_sh.at[segA.at[0]], sem).wait()
        return carry

    lax.fori_loop(0, _RPS // 8, cbody, 0)
    plsc.subcore_barrier()

    base = w * _RPW
    pltpu.sync_copy(src_hbm.at[pl.ds(base, _RPW)], srcB)
    pltpu.sync_copy(dst_hbm.at[pl.ds(base, _RPW)], dstA.at[pl.ds(0, _RPW)])
    pltpu.sync_copy(rt_hbm.at[pl.ds(base, _RPW)], rtA.at[pl.ds(0, _RPW)])

    def pbody(j, carry):
        for u in range(8):
            row = j * 8 + u
            for g in range(_C // _L):
                sl = (row, pl.ds(g * _L, _L))
                segA[sl] = dstA[sl] * _R + rtA[sl]
                gidxB[sl] = rtA[sl] * _N + srcB[sl]
            pltpu.async_copy(cnt_sh.at[segA.at[row]], cntrow.at[u], sem)
        for u in range(8):
            pltpu.make_async_copy(cnt_sh.at[segA.at[0]], cntrow.at[0],
                                  sem).wait()
        for u in range(8):
            row = j * 8 + u
            for g in range(_C // _L):
                c16 = cntrow[u, pl.ds(g * _L, _L)]
                scaleB[row, pl.ds(g * _L, _L)] = 1.0 / jnp.maximum(c16, 1.0)
        return carry

    lax.fori_loop(0, _RPW // 8, pbody, 0)
    pltpu.sync_copy(gidxB, gidx_hbm.at[pl.ds(base, _RPW)])
    pltpu.sync_copy(scaleB, scale_hbm.at[pl.ds(base, _RPW)])


def _edge_prep(srcr, dstr, rtr):
    return pl.kernel(
        _edge_prep_body,
        out_type=[jax.ShapeDtypeStruct((_ECP, _C), jnp.int32),
                  jax.ShapeDtypeStruct((_ECP, _C), jnp.float32)],
        mesh=_mesh,
        compiler_params=pltpu.CompilerParams(use_tc_tiling_on_sc=False),
        scratch_types=[
            pltpu.VMEM_SHARED((_CNTSZ,), jnp.float32),
            pltpu.VMEM((_RPS, _C), jnp.int32),
            pltpu.VMEM((_RPS, _C), jnp.int32),
            pltpu.VMEM((_RPS, _C), jnp.int32),
            pltpu.VMEM((_RPW, _C), jnp.int32),
            pltpu.VMEM((_RPW, _C), jnp.float32),
            pltpu.VMEM((_RPW, _C), jnp.int32),
            pltpu.VMEM((8, _C), jnp.float32),
            pltpu.VMEM((_C,), jnp.float32),
            pltpu.VMEM((_NRS,), jnp.float32),
            pltpu.SemaphoreType.DMA,
        ],
    )(srcr, dstr, rtr)


def _msg_body(xw_hbm, gidx_hbm, dst_hbm, scale_hbm,
              part_hbm,
              acc_sh, gidxT, dstT, scaleT,
              rows0, rows1, rows2, upd0, upd1,
              gs0, gs1, gs2, ss0, ss1):
    c = lax.axis_index("c")
    s = lax.axis_index("s")
    w = s * _NC + c
    base = w * _RPW
    rows = (rows0, rows1, rows2)
    upd = (upd0, upd1)
    gsem = (gs0, gs1, gs2)
    ssem = (ss0, ss1)

    def _gather(b, j):
        pltpu.async_copy(xw_hbm.at[gidxT.at[j]], rows[b], gsem[b])

    def _gwait(b):
        pltpu.make_async_copy(xw_hbm.at[gidxT.at[0]], rows[b], gsem[b]).wait()

    def _scatter(q, j):
        pltpu.async_copy(upd[q], acc_sh.at[dstT.at[j]], ssem[q], add=True)

    def _swait(q):
        pltpu.make_async_copy(upd[q], acc_sh.at[dstT.at[0]], ssem[q]).wait()

    def _widen(b, q, j):
        r = rows[b]
        o = upd[q]

        def grp(g, carry):
            scv = scaleT[j, pl.ds(g * _L, _L)]
            for i in range(_L):
                sc = scv[i]
                row = g * _L + i
                for u in range(_D // 32):
                    wv = r[row, pl.ds(u * _L, _L)]
                    lo = plsc.bitcast(wv << 16, jnp.float32)
                    hi = plsc.bitcast(wv & jnp.int32(-65536), jnp.float32)
                    o[row, pl.ds(u * 32, _L)] = lo * sc
                    o[row, pl.ds(u * 32 + _L, _L)] = hi * sc
            return carry

        lax.fori_loop(0, _C // _L, grp, 0)

    def zfill(j, carry):
        for u in range(_D // _L):
            upd0[j, pl.ds(u * _L, _L)] = jnp.zeros((_L,), jnp.float32)
        return carry

    lax.fori_loop(0, _C, zfill, 0)
    for k in range(_ZR // _C):
        pltpu.sync_copy(upd0, acc_sh.at[pl.ds(s * _ZR + k * _C, _C)])
    rem = _ZR % _C
    if rem:
        pltpu.sync_copy(upd0.at[pl.ds(0, rem)],
                        acc_sh.at[pl.ds(s * _ZR + (_ZR // _C) * _C, rem)])
    plsc.subcore_barrier()

    def sbody(sschunk, carry0):
        sbase = base + sschunk * _SS
        pltpu.sync_copy(gidx_hbm.at[pl.ds(sbase, _SS)], gidxT)
        pltpu.sync_copy(dst_hbm.at[pl.ds(sbase, _SS)], dstT)
        pltpu.sync_copy(scale_hbm.at[pl.ds(sbase, _SS)], scaleT)
        _gather(0, 0)
        _gather(1, 1)
        nt = (_SS - 2) // 6

        def tbody(t, carry):
            c0 = 6 * t
            for kb in range(6):
                j = c0 + kb
                b = kb % 3
                q = kb % 2
                _gwait(b)
                if kb < 2:
                    @pl.when(t > 0)
                    def _():
                        _swait(q)
                else:
                    _swait(q)
                _widen(b, q, j)
                _scatter(q, j)
                _gather((kb + 2) % 3, j + 2)
            return carry

        lax.fori_loop(0, nt, tbody, 0)
        for j in range(6 * nt, _SS):
            b = j % 3
            q = j % 2
            _gwait(b)
            _swait(q)
            _widen(b, q, j)
            _scatter(q, j)
            if j + 2 < _SS:
                _gather((j + 2) % 3, j + 2)
        for q in range(2):
            _swait(q)
        return carry0

    lax.fori_loop(0, _RPW // _SS, sbody, 0)
    plsc.subcore_barrier()
    pltpu.sync_copy(acc_sh.at[pl.ds(s * _ZR, _ZR)],
                    part_hbm.at[c].at[pl.ds(s * _ZR, _ZR)])


def _msg_pass(xw2d, gidx, dstr, scale):
    return pl.kernel(
        _msg_body,
        out_type=jax.ShapeDtypeStruct((_NC, _NACC, _D), jnp.float32),
        mesh=_mesh,
        compiler_params=pltpu.CompilerParams(use_tc_tiling_on_sc=False,
                                             needs_layout_passes=False),
        scratch_types=[
            pltpu.VMEM_SHARED((_NACC, _D), jnp.float32),
            pltpu.VMEM((_SS, _C), jnp.int32),
            pltpu.VMEM((_SS, _C), jnp.int32),
            pltpu.VMEM((_SS, _C), jnp.float32),
            pltpu.VMEM((_C, _D // 2), jnp.int32),
            pltpu.VMEM((_C, _D // 2), jnp.int32),
            pltpu.VMEM((_C, _D // 2), jnp.int32),
            pltpu.VMEM((_C, _D), jnp.float32),
            pltpu.VMEM((_C, _D), jnp.float32),
            pltpu.SemaphoreType.DMA,
            pltpu.SemaphoreType.DMA,
            pltpu.SemaphoreType.DMA,
            pltpu.SemaphoreType.DMA,
            pltpu.SemaphoreType.DMA,
        ],
    )(xw2d, gidx, dstr, scale)


def kernel(x, edge_index, edge_type, batch, basis, comp, root, bias):
    npad = _EP - _E
    ar = jnp.arange(npad, dtype=jnp.int32)
    src = jnp.concatenate([edge_index[0], (ar * 37) % _N]).reshape(_ECP, _C)
    dst = jnp.concatenate([edge_index[1],
                           _N + ar % (_NACC - _N)]).reshape(_ECP, _C)
    rtr = jnp.concatenate([edge_type,
                           jnp.zeros((npad,), jnp.int32)]).reshape(_ECP, _C)
    bias2d = bias.reshape(1, _D)
    batch2d = batch.reshape(_N, 1)

    wcat = _combine_weights(comp, basis)
    src32 = [(k // 2) if k % 2 == 0 else _L + (k - 1) // 2 for k in range(32)]
    perm = [r * _D + 32 * u + p
            for r in range(_R) for u in range(_D // 32) for p in src32]
    wcat = wcat[:, jnp.array(perm, dtype=jnp.int32)]
    gidx, scale = _edge_prep(src, dst, rtr)

    xw1, xr1 = _mm(x, wcat, root)
    xw1i = lax.bitcast_convert_type(
        xw1.reshape(_R * _N, _D // 2, 2), jnp.int32)
    part1 = _msg_pass(xw1i, gidx, dst, scale)
    h1, xw2, xr2 = _combine_relu_mm(part1, xr1, bias2d, wcat, root)
    xw2i = lax.bitcast_convert_type(
        xw2.reshape(_R * _N, _D // 2, 2), jnp.int32)
    part2 = _msg_pass(xw2i, gidx, dst, scale)
    h2, gf = _combine_pool(part2, xr2, bias2d, batch2d)
    return (gf, h1, h2)


# speedup vs baseline: 2.5256x; 2.5256x over previous
import jax
import jax.numpy as jnp
from jax import lax
from jax.experimental import pallas as pl
from jax.experimental.pallas import tpu as pltpu
from jax.experimental.pallas import tpu_sc as plsc

_N, _E, _D, _R, _NB, _G = 10000, 320000, 128, 8, 30, 64
_NC, _NS, _L = 2, 16, 16
_NW = _NC * _NS
_C = 64
_EP = 327680
_ECP = _EP // _C
_RPW = _ECP // _NW
_RPS = _ECP // _NS
_NACC = 10112
_ZR = _NACC // _NS
_CNTSZ = _NACC * _R
_NRS = _CNTSZ // _NS
_SS = 32
_BN = 1000

_mesh = plsc.VectorSubcoreMesh(core_axis_name="c", subcore_axis_name="s",
                               num_cores=_NC, num_subcores=_NS)


def _wcomb_body(comp_ref, basis_ref, out_ref):
    out_ref[...] = jnp.dot(comp_ref[...], basis_ref[...],
                           preferred_element_type=jnp.float32)


def _combine_weights(comp, basis):
    wfull = pl.pallas_call(
        _wcomb_body,
        out_shape=jax.ShapeDtypeStruct((_R, _D * _D), jnp.float32),
    )(comp, basis.reshape(_NB, _D * _D))
    return wfull.reshape(_R, _D, _D).transpose(1, 0, 2).reshape(_D, _R * _D)


def _mm_body(x_ref, wcat_ref, root_ref, xw_ref, xr_ref):
    xb = x_ref[...]
    xw_ref[...] = jnp.dot(xb, wcat_ref[...], preferred_element_type=jnp.float32)

    @pl.when(pl.program_id(1) == 0)
    def _():
        xr_ref[...] = jnp.dot(xb, root_ref[...],
                              preferred_element_type=jnp.float32)


def _mm(x, wcat, root):
    return pl.pallas_call(
        _mm_body,
        grid=(_N // _BN, _R),
        in_specs=[pl.BlockSpec((_BN, _D), lambda i, r: (i, 0)),
                  pl.BlockSpec((_D, _D), lambda i, r: (0, r)),
                  pl.BlockSpec((_D, _D), lambda i, r: (0, 0))],
        out_specs=[pl.BlockSpec((_BN, _D),
                                lambda i, r: (r * (_N // _BN) + i, 0)),
                   pl.BlockSpec((_BN, _D), lambda i, r: (i, 0))],
        out_shape=[jax.ShapeDtypeStruct((_R * _N, _D), jnp.float32),
                   jax.ShapeDtypeStruct((_N, _D), jnp.float32)],
    )(x, wcat, root)


def _comb_body(p_ref, xr_ref, b_ref, wcat_ref, root_ref,
               h_ref, xw_ref, xr2_ref, x2_ref):
    @pl.when(pl.program_id(1) == 0)
    def _():
        h = p_ref[0] + p_ref[1] + xr_ref[...] + b_ref[...]
        h_ref[...] = h
        x2 = jnp.maximum(h, 0.0)
        x2_ref[...] = x2
        xr2_ref[...] = jnp.dot(x2, root_ref[...],
                               preferred_element_type=jnp.float32)

    xw_ref[...] = jnp.dot(x2_ref[...], wcat_ref[...],
                          preferred_element_type=jnp.float32)


def _combine_relu_mm(part, xr, bias2d, wcat, root):
    return pl.pallas_call(
        _comb_body,
        grid=(_N // _BN, _R),
        in_specs=[pl.BlockSpec((_NC, _BN, _D), lambda i, r: (0, i, 0)),
                  pl.BlockSpec((_BN, _D), lambda i, r: (i, 0)),
                  pl.BlockSpec((1, _D), lambda i, r: (0, 0)),
                  pl.BlockSpec((_D, _D), lambda i, r: (0, r)),
                  pl.BlockSpec((_D, _D), lambda i, r: (0, 0))],
        out_specs=[pl.BlockSpec((_BN, _D), lambda i, r: (i, 0)),
                   pl.BlockSpec((_BN, _D),
                                lambda i, r: (r * (_N // _BN) + i, 0)),
                   pl.BlockSpec((_BN, _D), lambda i, r: (i, 0))],
        out_shape=[jax.ShapeDtypeStruct((_N, _D), jnp.float32),
                   jax.ShapeDtypeStruct((_R * _N, _D), jnp.float32),
                   jax.ShapeDtypeStruct((_N, _D), jnp.float32)],
        scratch_shapes=[pltpu.VMEM((_BN, _D), jnp.float32)],
    )(part, xr, bias2d, wcat, root)


def _final_body(p_ref, xr_ref, b_ref, bt_ref, h_ref, gf_ref):
    h = p_ref[0] + p_ref[1] + xr_ref[...] + b_ref[...]
    h_ref[...] = h
    gids = lax.broadcasted_iota(jnp.int32, (1, _G), 1)
    onehot = (bt_ref[...] == gids).astype(jnp.float32)
    contrib = lax.dot_general(onehot, h, (((0,), (0,)), ((), ())),
                              preferred_element_type=jnp.float32)

    @pl.when(pl.program_id(0) == 0)
    def _():
        gf_ref[...] = contrib

    @pl.when(pl.program_id(0) != 0)
    def _():
        gf_ref[...] += contrib


def _combine_pool(part, xr, bias2d, batch2d):
    return pl.pallas_call(
        _final_body,
        grid=(_N // _BN,),
        in_specs=[pl.BlockSpec((_NC, _BN, _D), lambda i: (0, i, 0)),
                  pl.BlockSpec((_BN, _D), lambda i: (i, 0)),
                  pl.BlockSpec((1, _D), lambda i: (0, 0)),
                  pl.BlockSpec((_BN, 1), lambda i: (i, 0))],
        out_specs=[pl.BlockSpec((_BN, _D), lambda i: (i, 0)),
                   pl.BlockSpec((_G, _D), lambda i: (0, 0))],
        out_shape=[jax.ShapeDtypeStruct((_N, _D), jnp.float32),
                   jax.ShapeDtypeStruct((_G, _D), jnp.float32)],
    )(part, xr, bias2d, batch2d)


def _edge_prep_body(src_hbm, dst_hbm, rt_hbm,
                    gidx_hbm, scale_hbm,
                    cnt_sh, dstA, rtA, segA, srcB, scaleB, gidxB,
                    cntrow, ones_v, zbuf, sem):
    c = lax.axis_index("c")
    s = lax.axis_index("s")
    w = s * _NC + c

    def zfill(j, carry):
        zbuf[pl.ds(j * _L, _L)] = jnp.zeros((_L,), jnp.float32)
        return carry

    lax.fori_loop(0, _NRS // _L, zfill, 0)
    pltpu.sync_copy(zbuf, cnt_sh.at[pl.ds(s * _NRS, _NRS)])
    pltpu.sync_copy(dst_hbm.at[pl.ds(s * _RPS, _RPS)], dstA)
    pltpu.sync_copy(rt_hbm.at[pl.ds(s * _RPS, _RPS)], rtA)
    for u in range(_C // _L):
        ones_v[pl.ds(u * _L, _L)] = jnp.full((_L,), 1.0, jnp.float32)
    plsc.subcore_barrier()

    def cbody(j, carry):
        for u in range(8):
            row = j * 8 + u
            for g in range(_C // _L):
                sl = (row, pl.ds(g * _L, _L))
                segA[sl] = dstA[sl] * _R + rtA[sl]
            pltpu.async_copy(ones_v, cnt_sh.at[segA.at[row]], sem, add=True)
        for u in range(8):
            pltpu.make_async_copy(ones_v, cnt_sh.at[segA.at[0]], sem).wait()
        return carry

    lax.fori_loop(0, _RPS // 8, cbody, 0)
    plsc.subcore_barrier()

    base = w * _RPW
    pltpu.sync_copy(src_hbm.at[pl.ds(base, _RPW)], srcB)
    pltpu.sync_copy(dst_hbm.at[pl.ds(base, _RPW)], dstA.at[pl.ds(0, _RPW)])
    pltpu.sync_copy(rt_hbm.at[pl.ds(base, _RPW)], rtA.at[pl.ds(0, _RPW)])

    def pbody(j, carry):
        for u in range(8):
            row = j * 8 + u
            for g in range(_C // _L):
                sl = (row, pl.ds(g * _L, _L))
                segA[sl] = dstA[sl] * _R + rtA[sl]
                gidxB[sl] = rtA[sl] * _N + srcB[sl]
            pltpu.async_copy(cnt_sh.at[segA.at[row]], cntrow.at[u], sem)
        for u in range(8):
            pltpu.make_async_copy(cnt_sh.at[segA.at[0]], cntrow.at[0],
                                  sem).wait()
        for u in range(8):
            row = j * 8 + u
            for g in range(_C // _L):
                c16 = cntrow[u, pl.ds(g * _L, _L)]
                scaleB[row, pl.ds(g * _L, _L)] = 1.0 / jnp.maximum(c16, 1.0)
        return carry

    lax.fori_loop(0, _RPW // 8, pbody, 0)
    pltpu.sync_copy(gidxB, gidx_hbm.at[pl.ds(base, _RPW)])
    pltpu.sync_copy(scaleB, scale_hbm.at[pl.ds(base, _RPW)])


def _edge_prep(srcr, dstr, rtr):
    return pl.kernel(
        _edge_prep_body,
        out_type=[jax.ShapeDtypeStruct((_ECP, _C), jnp.int32),
                  jax.ShapeDtypeStruct((_ECP, _C), jnp.float32)],
        mesh=_mesh,
        compiler_params=pltpu.CompilerParams(use_tc_tiling_on_sc=False),
        scratch_types=[
            pltpu.VMEM_SHARED((_CNTSZ,), jnp.float32),
            pltpu.VMEM((_RPS, _C), jnp.int32),
            pltpu.VMEM((_RPS, _C), jnp.int32),
            pltpu.VMEM((_RPS, _C), jnp.int32),
            pltpu.VMEM((_RPW, _C), jnp.int32),
            pltpu.VMEM((_RPW, _C), jnp.float32),
            pltpu.VMEM((_RPW, _C), jnp.int32),
            pltpu.VMEM((8, _C), jnp.float32),
            pltpu.VMEM((_C,), jnp.float32),
            pltpu.VMEM((_NRS,), jnp.float32),
            pltpu.SemaphoreType.DMA,
        ],
    )(srcr, dstr, rtr)


def _msg_body(xw_hbm, gidx_hbm, dst_hbm, scale_hbm,
              part_hbm,
              acc_sh, gidxT, dstT, scaleT,
              rows0, rows1, rows2, gs0, gs1, gs2, ss0, ss1, ss2):
    c = lax.axis_index("c")
    s = lax.axis_index("s")
    w = s * _NC + c
    base = w * _RPW
    rows = (rows0, rows1, rows2)
    gsem = (gs0, gs1, gs2)
    ssem = (ss0, ss1, ss2)

    def _gather(b, j):
        pltpu.async_copy(xw_hbm.at[gidxT.at[j]], rows[b], gsem[b])

    def _gwait(b):
        pltpu.make_async_copy(xw_hbm.at[gidxT.at[0]], rows[b], gsem[b]).wait()

    def _scatter(b, j):
        pltpu.async_copy(rows[b], acc_sh.at[dstT.at[j]], ssem[b], add=True)

    def _swait(b):
        pltpu.make_async_copy(rows[b], acc_sh.at[dstT.at[0]], ssem[b]).wait()

    def _scale(b, j):
        r = rows[b]

        def grp(g, carry):
            scv = scaleT[j, pl.ds(g * _L, _L)]
            for i in range(_L):
                sc = scv[i]
                for u in range(_D // _L):
                    sl = (g * _L + i, pl.ds(u * _L, _L))
                    r[sl] = r[sl] * sc
            return carry

        lax.fori_loop(0, _C // _L, grp, 0)

    def zfill(j, carry):
        for u in range(_D // _L):
            rows0[j, pl.ds(u * _L, _L)] = jnp.zeros((_L,), jnp.float32)
        return carry

    lax.fori_loop(0, _C, zfill, 0)
    for k in range(_ZR // _C):
        pltpu.sync_copy(rows0, acc_sh.at[pl.ds(s * _ZR + k * _C, _C)])
    rem = _ZR % _C
    if rem:
        pltpu.sync_copy(rows0.at[pl.ds(0, rem)],
                        acc_sh.at[pl.ds(s * _ZR + (_ZR // _C) * _C, rem)])
    plsc.subcore_barrier()

    for sschunk in range(_RPW // _SS):
        sbase = base + sschunk * _SS
        pltpu.sync_copy(gidx_hbm.at[pl.ds(sbase, _SS)], gidxT)
        pltpu.sync_copy(dst_hbm.at[pl.ds(sbase, _SS)], dstT)
        pltpu.sync_copy(scale_hbm.at[pl.ds(sbase, _SS)], scaleT)
        _gather(0, 0)
        _gather(1, 1)
        nt = (_SS - 2) // 3

        def tbody(t, carry):
            c0 = 3 * t
            for kb in range(3):
                j = c0 + kb
                _gwait(kb)
                _scale(kb, j)
                _scatter(kb, j)
                nb = (kb + 2) % 3
                if kb == 0:
                    @pl.when(t > 0)
                    def _():
                        _swait(nb)
                else:
                    _swait(nb)
                _gather(nb, j + 2)
            return carry

        lax.fori_loop(0, nt, tbody, 0)
        for j in range(3 * nt, _SS):
            b = j % 3
            _gwait(b)
            _scale(b, j)
            _scatter(b, j)
        for b in range(3):
            _swait(b)

    plsc.subcore_barrier()
    pltpu.sync_copy(acc_sh.at[pl.ds(s * _ZR, _ZR)],
                    part_hbm.at[c].at[pl.ds(s * _ZR, _ZR)])


def _msg_pass(xw2d, gidx, dstr, scale):
    return pl.kernel(
        _msg_body,
        out_type=jax.ShapeDtypeStruct((_NC, _NACC, _D), jnp.float32),
        mesh=_mesh,
        compiler_params=pltpu.CompilerParams(use_tc_tiling_on_sc=False),
        scratch_types=[
            pltpu.VMEM_SHARED((_NACC, _D), jnp.float32),
            pltpu.VMEM((_SS, _C), jnp.int32),
            pltpu.VMEM((_SS, _C), jnp.int32),
            pltpu.VMEM((_SS, _C), jnp.float32),
            pltpu.VMEM((_C, _D), jnp.float32),
            pltpu.VMEM((_C, _D), jnp.float32),
            pltpu.VMEM((_C, _D), jnp.float32),
            pltpu.SemaphoreType.DMA,
            pltpu.SemaphoreType.DMA,
            pltpu.SemaphoreType.DMA,
            pltpu.SemaphoreType.DMA,
            pltpu.SemaphoreType.DMA,
            pltpu.SemaphoreType.DMA,
        ],
    )(xw2d, gidx, dstr, scale)


def kernel(x, edge_index, edge_type, batch, basis, comp, root, bias):
    npad = _EP - _E
    ar = jnp.arange(npad, dtype=jnp.int32)
    src = jnp.concatenate([edge_index[0], (ar * 37) % _N]).reshape(_ECP, _C)
    dst = jnp.concatenate([edge_index[1],
                           _N + ar % (_NACC - _N)]).reshape(_ECP, _C)
    rtr = jnp.concatenate([edge_type,
                           jnp.zeros((npad,), jnp.int32)]).reshape(_ECP, _C)
    bias2d = bias.reshape(1, _D)
    batch2d = batch.reshape(_N, 1)

    wcat = _combine_weights(comp, basis)
    gidx, scale = _edge_prep(src, dst, rtr)

    xw1, xr1 = _mm(x, wcat, root)
    part1 = _msg_pass(xw1, gidx, dst, scale)
    h1, xw2, xr2 = _combine_relu_mm(part1, xr1, bias2d, wcat, root)
    part2 = _msg_pass(xw2, gidx, dst, scale)
    h2, gf = _combine_pool(part2, xr2, bias2d, batch2d)
    return (gf, h1, h2)
